# baseline (speedup 1.0000x reference)
import math

import jax
import jax.numpy as jnp
from jax.experimental import pallas as pl
from jax.experimental.pallas import tpu as pltpu

_TOPK = 2
_CAP_FACTOR = 2.0


def _moe_kernel(flat_ref, wg_ref, w1_ref, b1_ref, w2_ref, b2_ref, out_ref,
                g_scr, pos_scr, fbf_scr, p_scr, gath_scr, acc_scr, gv_scr):
    e = pl.program_id(0)
    k = pl.program_id(1)
    K = pl.num_programs(1)
    E, N = g_scr.shape
    cap = p_scr.shape[0]
    HC = w1_ref.shape[1]

    @pl.when((e == 0) & (k == 0))
    def _route():
        logits = jax.lax.dot_general(
            wg_ref[...], flat_ref[...],
            dimension_numbers=(((1,), (1,)), ((), ())),
            preferred_element_type=jnp.float32)
        m = jnp.max(logits, axis=0, keepdims=True)
        ex = jnp.exp(logits - m)
        g = ex / jnp.sum(ex, axis=0, keepdims=True)
        ioe = jax.lax.broadcasted_iota(jnp.int32, (E, N), 0)
        v1 = jnp.max(g, axis=0, keepdims=True)
        i1 = jnp.min(jnp.where(g == v1, ioe, E), axis=0, keepdims=True)
        m1 = ioe == i1
        g2 = jnp.where(m1, -jnp.inf, g)
        v2 = jnp.max(g2, axis=0, keepdims=True)
        i2 = jnp.min(jnp.where(g2 == v2, ioe, E), axis=0, keepdims=True)
        mask = (m1 | (ioe == i2)).astype(jnp.int32)
        cums = mask
        sh = 1
        while sh < N:
            shifted = jnp.concatenate(
                [jnp.zeros((E, sh), jnp.int32), cums[:, :N - sh]], axis=1)
            cums = cums + shifted
            sh *= 2
        pos_scr[...] = cums * mask - 1
        g_scr[...] = g
        fbf_scr[...] = flat_ref[...].astype(jnp.bfloat16)
        out_ref[...] = jnp.zeros_like(out_ref)

    @pl.when(k == 0)
    def _dispatch():
        pos_row = pos_scr[pl.ds(e, 1), :]
        ioc = jax.lax.broadcasted_iota(jnp.int32, (cap, N), 0)
        hit = ioc == pos_row
        p_scr[...] = hit.astype(jnp.bfloat16)
        g_row = g_scr[pl.ds(e, 1), :]
        gv_scr[...] = jnp.sum(
            jnp.where(hit, g_row, 0.0), axis=1, keepdims=True)
        gath_scr[...] = jax.lax.dot_general(
            p_scr[...], fbf_scr[...],
            dimension_numbers=(((1,), (0,)), ((), ())),
            preferred_element_type=jnp.float32).astype(jnp.bfloat16)
        acc_scr[...] = jnp.zeros_like(acc_scr)

    hpre = jax.lax.dot_general(
        gath_scr[...], w1_ref[0],
        dimension_numbers=(((1,), (1,)), ((), ())),
        preferred_element_type=jnp.float32)
    hpre = hpre + b1_ref[pl.ds(e, 1), pl.ds(k * HC, HC)]
    h = 0.5 * hpre * (1.0 + jax.lax.erf(hpre * (1.0 / math.sqrt(2.0))))
    acc_scr[...] += jax.lax.dot_general(
        h.astype(jnp.bfloat16), w2_ref[0],
        dimension_numbers=(((1,), (1,)), ((), ())),
        preferred_element_type=jnp.float32)

    @pl.when(k == K - 1)
    def _combine():
        outc = (acc_scr[...] + b2_ref[pl.ds(e, 1), :]) * gv_scr[...]
        out_ref[...] += jax.lax.dot_general(
            p_scr[...], outc.astype(jnp.bfloat16),
            dimension_numbers=(((0,), (0,)), ((), ())),
            preferred_element_type=jnp.float32)


def kernel(hidden_states, Wg, W1, b1, W2, b2):
    Bv, Tv, D = hidden_states.shape
    N = Bv * Tv
    E, H, _ = W1.shape
    cap = math.ceil(_CAP_FACTOR * N / E)
    HC = 2048
    K = H // HC

    flat = hidden_states.reshape(N, D)
    w1bf = W1.astype(jnp.bfloat16)
    w2bf = W2.astype(jnp.bfloat16)

    final = pl.pallas_call(
        _moe_kernel,
        grid=(E, K),
        in_specs=[
            pl.BlockSpec((N, D), lambda e, k: (0, 0)),
            pl.BlockSpec((E, D), lambda e, k: (0, 0)),
            pl.BlockSpec((1, HC, D), lambda e, k: (e, k, 0)),
            pl.BlockSpec((E, H), lambda e, k: (0, 0)),
            pl.BlockSpec((1, D, HC), lambda e, k: (e, 0, k)),
            pl.BlockSpec((E, D), lambda e, k: (0, 0)),
        ],
        out_specs=pl.BlockSpec((N, D), lambda e, k: (0, 0)),
        out_shape=jax.ShapeDtypeStruct((N, D), jnp.float32),
        scratch_shapes=[
            pltpu.VMEM((E, N), jnp.float32),
            pltpu.VMEM((E, N), jnp.int32),
            pltpu.VMEM((N, D), jnp.bfloat16),
            pltpu.VMEM((cap, N), jnp.bfloat16),
            pltpu.VMEM((cap, D), jnp.bfloat16),
            pltpu.VMEM((cap, D), jnp.float32),
            pltpu.VMEM((cap, 1), jnp.float32),
        ],
    )(flat, Wg, w1bf, b1, w2bf, b2)

    aux_loss = jnp.asarray(0.0, dtype=jnp.float32)
    return (final.reshape(Bv, Tv, D), aux_loss)


# speedup vs baseline: 1.6116x; 1.6116x over previous
import math

import jax
import jax.numpy as jnp
from jax.experimental import pallas as pl
from jax.experimental.pallas import tpu as pltpu

_TOPK = 2
_CAP_FACTOR = 2.0


def _router_kernel(flat_ref, wg_ref, p_ref, g_ref, gates_scr, pos_scr):
    e = pl.program_id(0)
    E, N = gates_scr.shape
    cap = p_ref.shape[1]

    @pl.when(e == 0)
    def _compute_routing():
        logits = jax.lax.dot_general(
            wg_ref[...], flat_ref[...],
            dimension_numbers=(((1,), (1,)), ((), ())),
            preferred_element_type=jnp.float32)
        m = jnp.max(logits, axis=0, keepdims=True)
        ex = jnp.exp(logits - m)
        g = ex / jnp.sum(ex, axis=0, keepdims=True)
        ioe = jax.lax.broadcasted_iota(jnp.int32, (E, N), 0)
        v1 = jnp.max(g, axis=0, keepdims=True)
        i1 = jnp.min(jnp.where(g == v1, ioe, E), axis=0, keepdims=True)
        m1 = ioe == i1
        g2 = jnp.where(m1, -jnp.inf, g)
        v2 = jnp.max(g2, axis=0, keepdims=True)
        i2 = jnp.min(jnp.where(g2 == v2, ioe, E), axis=0, keepdims=True)
        mask = (m1 | (ioe == i2)).astype(jnp.int32)
        cums = mask
        sh = 1
        while sh < N:
            shifted = jnp.concatenate(
                [jnp.zeros((E, sh), jnp.int32), cums[:, :N - sh]], axis=1)
            cums = cums + shifted
            sh *= 2
        pos_scr[...] = cums * mask - 1
        gates_scr[...] = g
        g_ref[...] = g

    pos_row = pos_scr[pl.ds(e, 1), :]
    ioc = jax.lax.broadcasted_iota(jnp.int32, (cap, N), 0)
    p_ref[0] = (ioc == pos_row).astype(jnp.float32)


def _expert_kernel(flat_ref, p_ref, g_ref, w1_ref, b1_ref, w2_ref, b2_ref,
                   out_ref, gath_scr, acc_scr):
    e = pl.program_id(0)
    k = pl.program_id(1)
    K = pl.num_programs(1)
    HC = w1_ref.shape[1]

    @pl.when((e == 0) & (k == 0))
    def _zero_out():
        out_ref[...] = jnp.zeros_like(out_ref)

    @pl.when(k == 0)
    def _dispatch():
        gath_scr[...] = jax.lax.dot_general(
            p_ref[0], flat_ref[...],
            dimension_numbers=(((1,), (0,)), ((), ())),
            preferred_element_type=jnp.float32)
        acc_scr[...] = jnp.zeros_like(acc_scr)

    hpre = jax.lax.dot_general(
        gath_scr[...], w1_ref[0],
        dimension_numbers=(((1,), (1,)), ((), ())),
        preferred_element_type=jnp.float32)
    hpre = hpre + b1_ref[pl.ds(e, 1), pl.ds(k * HC, HC)]
    h = 0.5 * hpre * (1.0 + jax.lax.erf(hpre * (1.0 / math.sqrt(2.0))))
    acc_scr[...] += jax.lax.dot_general(
        h, w2_ref[0],
        dimension_numbers=(((1,), (1,)), ((), ())),
        preferred_element_type=jnp.float32)

    @pl.when(k == K - 1)
    def _combine():
        gv = jax.lax.dot_general(
            p_ref[0], g_ref[pl.ds(e, 1), :],
            dimension_numbers=(((1,), (1,)), ((), ())),
            preferred_element_type=jnp.float32)
        outc = (acc_scr[...] + b2_ref[pl.ds(e, 1), :]) * gv
        out_ref[...] += jax.lax.dot_general(
            p_ref[0], outc,
            dimension_numbers=(((0,), (0,)), ((), ())),
            preferred_element_type=jnp.float32)


def kernel(hidden_states, Wg, W1, b1, W2, b2):
    Bv, Tv, D = hidden_states.shape
    N = Bv * Tv
    E, H, _ = W1.shape
    cap = math.ceil(_CAP_FACTOR * N / E)
    HC = 1024
    K = H // HC

    flat = hidden_states.reshape(N, D)

    p, g = pl.pallas_call(
        _router_kernel,
        grid=(E,),
        in_specs=[
            pl.BlockSpec((N, D), lambda e: (0, 0)),
            pl.BlockSpec((E, D), lambda e: (0, 0)),
        ],
        out_specs=[
            pl.BlockSpec((1, cap, N), lambda e: (e, 0, 0)),
            pl.BlockSpec((E, N), lambda e: (0, 0)),
        ],
        out_shape=[
            jax.ShapeDtypeStruct((E, cap, N), jnp.float32),
            jax.ShapeDtypeStruct((E, N), jnp.float32),
        ],
        scratch_shapes=[
            pltpu.VMEM((E, N), jnp.float32),
            pltpu.VMEM((E, N), jnp.int32),
        ],
    )(flat, Wg)

    final = pl.pallas_call(
        _expert_kernel,
        grid=(E, K),
        in_specs=[
            pl.BlockSpec((N, D), lambda e, k: (0, 0)),
            pl.BlockSpec((1, cap, N), lambda e, k: (e, 0, 0)),
            pl.BlockSpec((E, N), lambda e, k: (0, 0)),
            pl.BlockSpec((1, HC, D), lambda e, k: (e, k, 0)),
            pl.BlockSpec((E, H), lambda e, k: (0, 0)),
            pl.BlockSpec((1, D, HC), lambda e, k: (e, 0, k)),
            pl.BlockSpec((E, D), lambda e, k: (0, 0)),
        ],
        out_specs=pl.BlockSpec((N, D), lambda e, k: (0, 0)),
        out_shape=jax.ShapeDtypeStruct((N, D), jnp.float32),
        scratch_shapes=[
            pltpu.VMEM((cap, D), jnp.float32),
            pltpu.VMEM((cap, D), jnp.float32),
        ],
    )(flat, p, g, W1, b1, W2, b2)

    aux_loss = jnp.asarray(0.0, dtype=jnp.float32)
    return (final.reshape(Bv, Tv, D), aux_loss)


# speedup vs baseline: 1.6318x; 1.0125x over previous
import math

import jax
import jax.numpy as jnp
from jax.experimental import pallas as pl
from jax.experimental.pallas import tpu as pltpu

_TOPK = 2
_CAP_FACTOR = 2.0


def _router_kernel(flat_ref, wg_ref, p_ref, g_ref, gates_scr, pos_scr):
    e = pl.program_id(0)
    E, N = gates_scr.shape
    cap = p_ref.shape[1]

    @pl.when(e == 0)
    def _compute_routing():
        logits = jax.lax.dot_general(
            wg_ref[...], flat_ref[...],
            dimension_numbers=(((1,), (1,)), ((), ())),
            preferred_element_type=jnp.float32)
        m = jnp.max(logits, axis=0, keepdims=True)
        ex = jnp.exp(logits - m)
        g = ex / jnp.sum(ex, axis=0, keepdims=True)
        ioe = jax.lax.broadcasted_iota(jnp.int32, (E, N), 0)
        v1 = jnp.max(g, axis=0, keepdims=True)
        i1 = jnp.min(jnp.where(g == v1, ioe, E), axis=0, keepdims=True)
        m1 = ioe == i1
        g2 = jnp.where(m1, -jnp.inf, g)
        v2 = jnp.max(g2, axis=0, keepdims=True)
        i2 = jnp.min(jnp.where(g2 == v2, ioe, E), axis=0, keepdims=True)
        mask = (m1 | (ioe == i2)).astype(jnp.int32)
        cums = mask
        sh = 1
        while sh < N:
            shifted = jnp.concatenate(
                [jnp.zeros((E, sh), jnp.int32), cums[:, :N - sh]], axis=1)
            cums = cums + shifted
            sh *= 2
        pos_scr[...] = cums * mask - 1
        gates_scr[...] = g
        g_ref[...] = g

    pos_row = pos_scr[pl.ds(e, 1), :]
    ioc = jax.lax.broadcasted_iota(jnp.int32, (cap, N), 0)
    p_ref[0] = (ioc == pos_row).astype(jnp.bfloat16)


def _expert_kernel(flat_ref, p_ref, g_ref, w1_ref, b1_ref, w2_ref, b2_ref,
                   out_ref, fbf_scr, gath_scr, acc_scr, gv_scr):
    e = pl.program_id(0)
    k = pl.program_id(1)
    K = pl.num_programs(1)
    HC = w1_ref.shape[1]

    @pl.when((e == 0) & (k == 0))
    def _zero_out():
        out_ref[...] = jnp.zeros_like(out_ref)
        fbf_scr[...] = flat_ref[...].astype(jnp.bfloat16)

    @pl.when(k == 0)
    def _dispatch():
        gath_scr[...] = jax.lax.dot_general(
            p_ref[0], fbf_scr[...],
            dimension_numbers=(((1,), (0,)), ((), ())),
            preferred_element_type=jnp.float32).astype(jnp.bfloat16)
        acc_scr[...] = jnp.zeros_like(acc_scr)
        pf = p_ref[0].astype(jnp.float32)
        gv_scr[...] = jnp.sum(
            pf * g_ref[pl.ds(e, 1), :], axis=1, keepdims=True)

    hpre = jax.lax.dot_general(
        gath_scr[...], w1_ref[0].astype(jnp.bfloat16),
        dimension_numbers=(((1,), (1,)), ((), ())),
        preferred_element_type=jnp.float32)
    hpre = hpre + b1_ref[pl.ds(e, 1), pl.ds(k * HC, HC)]
    h = 0.5 * hpre * (1.0 + jax.lax.erf(hpre * (1.0 / math.sqrt(2.0))))
    acc_scr[...] += jax.lax.dot_general(
        h.astype(jnp.bfloat16), w2_ref[0].astype(jnp.bfloat16),
        dimension_numbers=(((1,), (1,)), ((), ())),
        preferred_element_type=jnp.float32)

    @pl.when(k == K - 1)
    def _combine():
        outc = (acc_scr[...] + b2_ref[pl.ds(e, 1), :]) * gv_scr[...]
        out_ref[...] += jax.lax.dot_general(
            p_ref[0], outc.astype(jnp.bfloat16),
            dimension_numbers=(((0,), (0,)), ((), ())),
            preferred_element_type=jnp.float32)


def kernel(hidden_states, Wg, W1, b1, W2, b2):
    Bv, Tv, D = hidden_states.shape
    N = Bv * Tv
    E, H, _ = W1.shape
    cap = math.ceil(_CAP_FACTOR * N / E)
    HC = 1024
    K = H // HC

    flat = hidden_states.reshape(N, D)

    p, g = pl.pallas_call(
        _router_kernel,
        grid=(E,),
        in_specs=[
            pl.BlockSpec((N, D), lambda e: (0, 0)),
            pl.BlockSpec((E, D), lambda e: (0, 0)),
        ],
        out_specs=[
            pl.BlockSpec((1, cap, N), lambda e: (e, 0, 0)),
            pl.BlockSpec((E, N), lambda e: (0, 0)),
        ],
        out_shape=[
            jax.ShapeDtypeStruct((E, cap, N), jnp.bfloat16),
            jax.ShapeDtypeStruct((E, N), jnp.float32),
        ],
        scratch_shapes=[
            pltpu.VMEM((E, N), jnp.float32),
            pltpu.VMEM((E, N), jnp.int32),
        ],
    )(flat, Wg)

    final = pl.pallas_call(
        _expert_kernel,
        grid=(E, K),
        in_specs=[
            pl.BlockSpec((N, D), lambda e, k: (0, 0)),
            pl.BlockSpec((1, cap, N), lambda e, k: (e, 0, 0)),
            pl.BlockSpec((E, N), lambda e, k: (0, 0)),
            pl.BlockSpec((1, HC, D), lambda e, k: (e, k, 0)),
            pl.BlockSpec((E, H), lambda e, k: (0, 0)),
            pl.BlockSpec((1, D, HC), lambda e, k: (e, 0, k)),
            pl.BlockSpec((E, D), lambda e, k: (0, 0)),
        ],
        out_specs=pl.BlockSpec((N, D), lambda e, k: (0, 0)),
        out_shape=jax.ShapeDtypeStruct((N, D), jnp.float32),
        scratch_shapes=[
            pltpu.VMEM((N, D), jnp.bfloat16),
            pltpu.VMEM((cap, D), jnp.bfloat16),
            pltpu.VMEM((cap, D), jnp.float32),
            pltpu.VMEM((cap, 1), jnp.float32),
        ],
    )(flat, p, g, W1, b1, W2, b2)

    aux_loss = jnp.asarray(0.0, dtype=jnp.float32)
    return (final.reshape(Bv, Tv, D), aux_loss)
